# Initial kernel scaffold; baseline (speedup 1.0000x reference)
import jax
import jax.numpy as jnp
from jax.experimental import pallas as pl


def kernel(features, edge_index, edge_weight, W, bias, prelu_a):
    raise NotImplementedError("write your pallas kernel here")


# speedup vs baseline: 3.8701x; 3.8701x over previous
import functools

import jax
import jax.numpy as jnp
from jax import lax
from jax.experimental import pallas as pl
from jax.experimental.pallas import tpu as pltpu
from jax.experimental.pallas import tpu_sc as plsc

N_NODES = 10000
N_EDGES = 160000
D_IN = 256
D_OUT = 256

NC = 2
NS = 16
DH = D_OUT // NC

CHUNK = 128
NCHUNKS = 80
EDGES_PER_TILE = NCHUNKS * CHUNK
N_EDGES_PAD = NS * EDGES_PER_TILE
N_NODES_PAD = 10240
NODES_PER_TILE = N_NODES_PAD // NS
OUT_CHUNK = 128
N_OUT_CHUNKS = NODES_PER_TILE // OUT_CHUNK
FVECS = DH // 16


def _mm_body(x_ref, w_ref, h_ref, ht_ref):
    x = x_ref[...]
    w = w_ref[...]
    h = lax.dot_general(x, w, (((1,), (1,)), ((), ())),
                        preferred_element_type=jnp.float32)
    h_ref[...] = h
    ht_ref[0] = h[:, :DH]
    ht_ref[1] = h[:, DH:]


def _matmul(x, w):
    m_blk = 2000
    grid = (N_NODES // m_blk,)
    return pl.pallas_call(
        _mm_body,
        grid=grid,
        in_specs=[
            pl.BlockSpec((m_blk, D_IN), lambda i: (i, 0)),
            pl.BlockSpec((D_OUT, D_IN), lambda i: (0, 0)),
        ],
        out_specs=[
            pl.BlockSpec((m_blk, D_OUT), lambda i: (i, 0)),
            pl.BlockSpec((NC, m_blk, DH), lambda i: (0, i, 0)),
        ],
        out_shape=[
            jax.ShapeDtypeStruct((N_NODES, D_OUT), jnp.float32),
            jax.ShapeDtypeStruct((NC, N_NODES, DH), jnp.float32),
        ],
    )(x, w)


MBLK = 8
N_MBLK = NCHUNKS // MBLK


def _sc_agg_body(ht_hbm, row_hbm, col_hbm, w_hbm, bias_hbm, a_hbm, act_hbm,
                 agg, colb, rowb, wb, gbuf, bias_v, a_v, sem):
    c = lax.axis_index("c")
    s = lax.axis_index("s")
    ht_c = ht_hbm.at[c]

    pltpu.sync_copy(bias_hbm.at[c], bias_v)
    pltpu.sync_copy(a_hbm, a_v)

    def _zrow(r, _):
        for f in range(FVECS):
            gbuf[r, pl.ds(f * 16, 16)] = jnp.zeros((16,), jnp.float32)
        return 0
    lax.fori_loop(0, OUT_CHUNK, _zrow, 0)
    def _zcopy(j, _):
        pltpu.sync_copy(gbuf, agg.at[pl.ds(s * NODES_PER_TILE + j * OUT_CHUNK,
                                           OUT_CHUNK)])
        return 0
    lax.fori_loop(0, N_OUT_CHUNKS, _zcopy, 0)
    plsc.subcore_barrier()

    def _mblock(b, _):
        sl_b = pl.ds(b * MBLK, MBLK)
        pltpu.sync_copy(row_hbm.at[s].at[sl_b], rowb)
        pltpu.sync_copy(col_hbm.at[s].at[sl_b], colb)
        pltpu.sync_copy(w_hbm.at[s].at[sl_b], wb)

        def _chunk(k, _):
            pltpu.async_copy(ht_c.at[colb.at[k]], gbuf, sem).wait()

            def _scale(g, _):
                wvec = wb[k, pl.ds(g * 16, 16)]
                for e in range(16):
                    w = wvec[e]
                    r = g * 16 + e
                    for f in range(FVECS):
                        sl = pl.ds(f * 16, 16)
                        gbuf[r, sl] = gbuf[r, sl] * w
                return 0
            lax.fori_loop(0, CHUNK // 16, _scale, 0)

            pltpu.sync_copy(gbuf, agg.at[rowb.at[k]], add=True)
            return 0
        lax.fori_loop(0, MBLK, _chunk, 0)
        return 0
    lax.fori_loop(0, N_MBLK, _mblock, 0)
    plsc.subcore_barrier()

    act_c = act_hbm.at[c]
    def _out(j, _):
        base = s * NODES_PER_TILE + j * OUT_CHUNK
        pltpu.sync_copy(agg.at[pl.ds(base, OUT_CHUNK)], gbuf)

        def _prelu(r, _):
            for f in range(FVECS):
                sl = pl.ds(f * 16, 16)
                v = gbuf[r, sl] + bias_v[0, sl]
                a = a_v[...]
                gbuf[r, sl] = jnp.where(v >= 0, v, a * v)
            return 0
        lax.fori_loop(0, OUT_CHUNK, _prelu, 0)

        pltpu.sync_copy(gbuf, act_c.at[pl.ds(base, OUT_CHUNK)])
        return 0
    lax.fori_loop(0, N_OUT_CHUNKS, _out, 0)


_sc_agg = functools.partial(
    pl.kernel,
    out_type=jax.ShapeDtypeStruct((NC, N_NODES_PAD, DH), jnp.float32),
    mesh=plsc.VectorSubcoreMesh(core_axis_name="c", subcore_axis_name="s"),
    scratch_types=[
        pltpu.VMEM_SHARED((N_NODES_PAD, DH), jnp.float32),
        pltpu.VMEM((MBLK, CHUNK), jnp.int32),
        pltpu.VMEM((MBLK, CHUNK), jnp.int32),
        pltpu.VMEM((MBLK, CHUNK), jnp.float32),
        pltpu.VMEM((CHUNK, DH), jnp.float32),
        pltpu.VMEM((8, DH), jnp.float32),
        pltpu.VMEM((16,), jnp.float32),
        pltpu.SemaphoreType.DMA,
    ],
)(_sc_agg_body)


@jax.jit
def kernel(features, edge_index, edge_weight, W, bias, prelu_a):
    x = features.reshape(N_NODES, D_IN)
    h, ht = _matmul(x, W)

    npad = N_EDGES_PAD - N_EDGES
    pad_idx = (jnp.arange(npad, dtype=jnp.int32) * 37) % N_NODES
    row = jnp.concatenate([edge_index[0].astype(jnp.int32), pad_idx])
    col = jnp.concatenate([edge_index[1].astype(jnp.int32), pad_idx])
    ew = jnp.concatenate([edge_weight.astype(jnp.float32),
                          jnp.zeros((npad,), jnp.float32)])
    row = row.reshape(NS, NCHUNKS, CHUNK)
    col = col.reshape(NS, NCHUNKS, CHUNK)
    ew = ew.reshape(NS, NCHUNKS, CHUNK)
    bias2 = jnp.broadcast_to(bias.reshape(NC, 1, DH), (NC, 8, DH))
    a16 = jnp.broadcast_to(prelu_a.astype(jnp.float32), (16,))

    act2 = _sc_agg(ht, row, col, ew, bias2, a16)
    act = jnp.moveaxis(act2[:, :N_NODES], 0, 1).reshape(1, N_NODES, D_OUT)
    hidden = h.reshape(1, N_NODES, D_OUT)
    return (act, hidden)
